# baseline (speedup 1.0000x reference)
import functools

import jax
import jax.numpy as jnp
from jax import lax
from jax.experimental import pallas as pl
from jax.experimental.pallas import tpu as pltpu
from jax.experimental.pallas import tpu_sc as plsc

_NC = 2
_NS = 16
_LANES = 16
_CH = 128
_NBUF = 2


def _pad_chunks(e):
    per_tile = -(-e // (_CH * _NC * _NS))
    per_tile = -(-per_tile // 8) * 8
    n_chunks = per_tile * _NC * _NS
    return n_chunks, per_tile


def _row_split(n_nodes):
    rows_main = (n_nodes // _NS) // 8 * 8
    rows_tail = n_nodes - rows_main * _NS
    assert rows_tail % 8 == 0
    return rows_main, rows_tail


def _sc_degree(dst2, n_nodes):
    n_chunks = dst2.shape[0]
    per_tile = n_chunks // (_NC * _NS)
    rows_main, rows_tail = _row_split(n_nodes)
    zn = rows_main + rows_tail
    n_acc = n_nodes + _LANES

    mesh = plsc.VectorSubcoreMesh(core_axis_name="c", subcore_axis_name="s")

    @functools.partial(
        pl.kernel,
        out_type=(jax.ShapeDtypeStruct((n_nodes,), jnp.float32),
                  jax.ShapeDtypeStruct((n_nodes,), jnp.float32)),
        mesh=mesh,
        scratch_types=[
            pltpu.VMEM((per_tile, _CH), jnp.int32),
            pltpu.VMEM((_CH,), jnp.float32),
            pltpu.VMEM((zn,), jnp.float32),
            pltpu.VMEM_SHARED((n_acc,), jnp.float32),
            pltpu.SemaphoreType.DMA,
        ],
    )
    def k(dst_hbm, out0_hbm, out1_hbm, idx_v, ones_v, zero_v, acc_sh, sem):
        cid = lax.axis_index("c")
        sid = lax.axis_index("s")

        one16 = jnp.full((_LANES,), 1.0, jnp.float32)
        zero16 = jnp.zeros((_LANES,), jnp.float32)

        @pl.loop(0, _CH // _LANES)
        def _(i):
            ones_v[pl.ds(i * _LANES, _LANES)] = one16

        @pl.loop(0, zn // _LANES)
        def _(i):
            zero_v[pl.ds(i * _LANES, _LANES)] = zero16

        c0 = (cid * _NS + sid) * per_tile
        pltpu.sync_copy(dst_hbm.at[pl.ds(c0, per_tile)], idx_v)

        row0 = sid * rows_main
        pltpu.sync_copy(zero_v.at[pl.ds(0, rows_main)],
                        acc_sh.at[pl.ds(row0, rows_main)])

        @pl.when(sid == _NS - 1)
        def _():
            pltpu.sync_copy(zero_v.at[pl.ds(0, rows_tail)],
                            acc_sh.at[pl.ds(_NS * rows_main, rows_tail)])
            pltpu.sync_copy(zero_v.at[pl.ds(0, _LANES)],
                            acc_sh.at[pl.ds(n_nodes, _LANES)])

        plsc.subcore_barrier()

        wave = 8
        assert per_tile % wave == 0

        @pl.loop(0, per_tile // wave)
        def _(w):
            for j in range(wave):
                pltpu.async_copy(ones_v, acc_sh.at[idx_v.at[w * wave + j]],
                                 sem, add=True)
            for j in range(wave):
                pltpu.make_async_copy(
                    ones_v, acc_sh.at[idx_v.at[0]], sem).wait()

        plsc.subcore_barrier()

        def copy_out(out_hbm):
            pltpu.sync_copy(acc_sh.at[pl.ds(row0, rows_main)],
                            zero_v.at[pl.ds(0, rows_main)])
            pltpu.sync_copy(zero_v.at[pl.ds(0, rows_main)],
                            out_hbm.at[pl.ds(row0, rows_main)])

            @pl.when(sid == _NS - 1)
            def _():
                pltpu.sync_copy(
                    acc_sh.at[pl.ds(_NS * rows_main, rows_tail)],
                    zero_v.at[pl.ds(rows_main, rows_tail)],
                )
                pltpu.sync_copy(
                    zero_v.at[pl.ds(rows_main, rows_tail)],
                    out_hbm.at[pl.ds(_NS * rows_main, rows_tail)],
                )

        @pl.when(cid == 0)
        def _():
            copy_out(out0_hbm)

        @pl.when(cid == 1)
        def _():
            copy_out(out1_hbm)

    return k(dst2)


def _sc_aggregate(g, src_p, dst_p, zeros2, n_nodes):
    d = g.shape[1]
    e_pad = src_p.shape[0]
    n_chunks = e_pad // _CH
    per_tile = n_chunks // (_NC * _NS)
    assert per_tile % _NBUF == 0
    rows_main, rows_tail = _row_split(n_nodes)
    n_acc = n_nodes + 8

    mesh = plsc.VectorSubcoreMesh(core_axis_name="c", subcore_axis_name="s")

    @functools.partial(
        pl.kernel,
        out_type=jax.ShapeDtypeStruct((_NC, n_nodes, d), jnp.float32),
        mesh=mesh,
        scratch_types=[
            pltpu.VMEM((_NBUF, _CH), jnp.int32),
            pltpu.VMEM((_NBUF, _CH), jnp.int32),
            pltpu.VMEM((_NBUF * _CH, d), jnp.float32),
            pltpu.VMEM_SHARED((n_acc, d), jnp.float32),
            [pltpu.SemaphoreType.DMA] * _NBUF,
            [pltpu.SemaphoreType.DMA] * _NBUF,
            [pltpu.SemaphoreType.DMA] * _NBUF,
        ],
    )
    def k(g_hbm, src_hbm, dst_hbm, zeros_hbm, out_hbm, sidx_v, didx_v,
          rows_v, acc_sh, sem_g, sem_s, sem_i):
        cid = lax.axis_index("c")
        sid = lax.axis_index("s")

        c0 = (cid * _NS + sid) * per_tile

        def load_idx(t, j, sem):
            base = (c0 + t) * _CH
            pltpu.async_copy(src_hbm.at[pl.ds(base, _CH)], sidx_v.at[j], sem)
            pltpu.async_copy(dst_hbm.at[pl.ds(base, _CH)], didx_v.at[j], sem)

        def wait_idx(j, sem):
            pltpu.make_async_copy(
                src_hbm.at[pl.ds(0, _CH)], sidx_v.at[j], sem).wait()
            pltpu.make_async_copy(
                dst_hbm.at[pl.ds(0, _CH)], didx_v.at[j], sem).wait()

        pltpu.sync_copy(zeros_hbm, rows_v.at[pl.ds(0, _CH)])
        row0 = sid * rows_main
        nz_full = rows_main // _CH
        z_rem = rows_main - nz_full * _CH

        @pl.loop(0, nz_full)
        def _(j):
            pltpu.sync_copy(rows_v.at[pl.ds(0, _CH)],
                            acc_sh.at[pl.ds(row0 + j * _CH, _CH)])

        pltpu.sync_copy(rows_v.at[pl.ds(0, z_rem)],
                        acc_sh.at[pl.ds(row0 + nz_full * _CH, z_rem)])

        @pl.when(sid == _NS - 1)
        def _():
            pltpu.sync_copy(
                rows_v.at[pl.ds(0, rows_tail + 8)],
                acc_sh.at[pl.ds(_NS * rows_main, rows_tail + 8)],
            )

        for j in range(_NBUF):
            load_idx(j, j, sem_i[j])
        for j in range(_NBUF):
            wait_idx(j, sem_i[j])
            pltpu.async_copy(g_hbm.at[sidx_v.at[j]], rows_v.at[pl.ds(j * _CH, _CH)], sem_g[j])

        plsc.subcore_barrier()

        @pl.loop(0, per_tile // _NBUF)
        def _(i):
            t = i * _NBUF
            for j in range(_NBUF):
                pltpu.make_async_copy(
                    g_hbm.at[sidx_v.at[0]], rows_v.at[pl.ds(j * _CH, _CH)], sem_g[j]).wait()
                pltpu.async_copy(rows_v.at[pl.ds(j * _CH, _CH)], acc_sh.at[didx_v.at[j]],
                                 sem_s[j], add=True)
            for j in range(_NBUF):
                pltpu.make_async_copy(
                    rows_v.at[pl.ds(j * _CH, _CH)], acc_sh.at[didx_v.at[0]], sem_s[j]).wait()

                @pl.when(t + _NBUF + j < per_tile)
                def _():
                    load_idx(t + _NBUF + j, j, sem_i[j])

            for j in range(_NBUF):
                @pl.when(t + _NBUF + j < per_tile)
                def _():
                    wait_idx(j, sem_i[j])
                    pltpu.async_copy(g_hbm.at[sidx_v.at[j]], rows_v.at[pl.ds(j * _CH, _CH)],
                                     sem_g[j])

        plsc.subcore_barrier()
        pltpu.sync_copy(
            acc_sh.at[pl.ds(row0, rows_main)],
            out_hbm.at[cid].at[pl.ds(row0, rows_main)],
        )

        @pl.when(sid == _NS - 1)
        def _():
            pltpu.sync_copy(
                acc_sh.at[pl.ds(_NS * rows_main, rows_tail)],
                out_hbm.at[cid].at[pl.ds(_NS * rows_main, rows_tail)],
            )

    return k(g, src_p, dst_p, zeros2)


def _dinv_from_parts(d0_ref, d1_ref):
    deg = d0_ref[...] + d1_ref[...]
    return jnp.where(deg > 0, lax.rsqrt(jnp.maximum(deg, 1e-12)), 0.0)


def _tc_prep(x, w, d0c, d1c):
    n, d = x.shape
    br = 2000
    assert n % br == 0

    def body(x_ref, w_ref, d0_ref, d1_ref, g_ref):
        dinv = _dinv_from_parts(d0_ref, d1_ref)
        h = jnp.dot(x_ref[...], w_ref[...], preferred_element_type=jnp.float32)
        g_ref[...] = h * dinv

    return pl.pallas_call(
        body,
        grid=(n // br,),
        in_specs=[
            pl.BlockSpec((br, d), lambda i: (i, 0)),
            pl.BlockSpec((d, d), lambda i: (0, 0)),
            pl.BlockSpec((br, 1), lambda i: (i, 0)),
            pl.BlockSpec((br, 1), lambda i: (i, 0)),
        ],
        out_specs=pl.BlockSpec((br, d), lambda i: (i, 0)),
        out_shape=jax.ShapeDtypeStruct((n, d), jnp.float32),
    )(x, w, d0c, d1c)


def _tc_finish(accp, d0c, d1c, b2):
    n, d = accp.shape[1], accp.shape[2]
    br = 2000
    assert n % br == 0

    def body(a_ref, d0_ref, d1_ref, b_ref, o_ref):
        dinv = _dinv_from_parts(d0_ref, d1_ref)
        o_ref[...] = (a_ref[0] + a_ref[1]) * dinv + b_ref[...]

    return pl.pallas_call(
        body,
        grid=(n // br,),
        in_specs=[
            pl.BlockSpec((_NC, br, d), lambda i: (0, i, 0)),
            pl.BlockSpec((br, 1), lambda i: (i, 0)),
            pl.BlockSpec((br, 1), lambda i: (i, 0)),
            pl.BlockSpec((1, d), lambda i: (0, 0)),
        ],
        out_specs=pl.BlockSpec((br, d), lambda i: (i, 0)),
        out_shape=jax.ShapeDtypeStruct((n, d), jnp.float32),
    )(accp, d0c, d1c, b2)


def kernel(x, edge_index, W, b):
    n, d = x.shape
    e = edge_index.shape[1]
    ei = edge_index.astype(jnp.int32)
    n_chunks, _ = _pad_chunks(e)
    pad = n_chunks * _CH - e
    src_p = jnp.concatenate([ei[0], jnp.zeros((pad,), jnp.int32)])
    dst_p = jnp.concatenate([ei[1], jnp.full((pad,), n, jnp.int32)])
    dst2 = dst_p.reshape(n_chunks, _CH)
    zeros2 = jnp.zeros((_CH, d), jnp.float32)

    deg0, deg1 = _sc_degree(dst2, n)
    d0c = deg0.reshape(n, 1)
    d1c = deg1.reshape(n, 1)
    g = _tc_prep(x, W, d0c, d1c)
    accp = _sc_aggregate(g, src_p, dst_p, zeros2, n)
    return _tc_finish(accp, d0c, d1c, b.reshape(1, d))


# speedup vs baseline: 2.7606x; 2.7606x over previous
import functools

import jax
import jax.numpy as jnp
from jax import lax
from jax.experimental import pallas as pl
from jax.experimental.pallas import tpu as pltpu
from jax.experimental.pallas import tpu_sc as plsc

_NC = 2
_NS = 16
_LANES = 16
_CH = 128
_NBUF = 2


def _pad_chunks(e):
    per_tile = -(-e // (_CH * _NC * _NS))
    per_tile = -(-per_tile // 8) * 8
    n_chunks = per_tile * _NC * _NS
    return n_chunks, per_tile


def _row_split(n_nodes):
    rows_main = (n_nodes // _NS) // 8 * 8
    rows_tail = n_nodes - rows_main * _NS
    assert rows_tail % 8 == 0
    return rows_main, rows_tail


def _sc_degree(dst2, n_nodes, e_real):
    n_chunks = dst2.shape[0]
    per_tile = n_chunks // (_NC * _NS)
    rows_main, rows_tail = _row_split(n_nodes)
    zn = rows_main + rows_tail
    n_acc = n_nodes + _LANES

    mesh = plsc.VectorSubcoreMesh(core_axis_name="c", subcore_axis_name="s")

    @functools.partial(
        pl.kernel,
        out_type=(jax.ShapeDtypeStruct((n_nodes,), jnp.float32),
                  jax.ShapeDtypeStruct((n_nodes,), jnp.float32)),
        mesh=mesh,
        scratch_types=[
            pltpu.VMEM((per_tile, _CH), jnp.int32),
            pltpu.VMEM((_CH,), jnp.float32),
            pltpu.VMEM((zn,), jnp.float32),
            pltpu.VMEM_SHARED((n_acc,), jnp.float32),
            pltpu.SemaphoreType.DMA,
        ],
    )
    def k(dst_hbm, out0_hbm, out1_hbm, idx_v, ones_v, zero_v, acc_sh, sem):
        cid = lax.axis_index("c")
        sid = lax.axis_index("s")

        one16 = jnp.full((_LANES,), 1.0, jnp.float32)
        zero16 = jnp.zeros((_LANES,), jnp.float32)

        @pl.loop(0, _CH // _LANES)
        def _(i):
            ones_v[pl.ds(i * _LANES, _LANES)] = one16

        @pl.loop(0, zn // _LANES)
        def _(i):
            zero_v[pl.ds(i * _LANES, _LANES)] = zero16

        c0 = (cid * _NS + sid) * per_tile
        pltpu.sync_copy(dst_hbm.at[pl.ds(c0, per_tile)], idx_v)

        row0 = sid * rows_main
        pltpu.sync_copy(zero_v.at[pl.ds(0, rows_main)],
                        acc_sh.at[pl.ds(row0, rows_main)])

        @pl.when(sid == _NS - 1)
        def _():
            pltpu.sync_copy(zero_v.at[pl.ds(0, rows_tail)],
                            acc_sh.at[pl.ds(_NS * rows_main, rows_tail)])
            pltpu.sync_copy(zero_v.at[pl.ds(0, _LANES)],
                            acc_sh.at[pl.ds(n_nodes, _LANES)])

        plsc.subcore_barrier()

        wave = 8
        assert per_tile % wave == 0
        n_eff = -(-e_real // _CH)

        @pl.loop(0, per_tile // wave)
        def _(w):
            for j in range(wave):
                @pl.when(c0 + w * wave + j < n_eff)
                def _():
                    pltpu.async_copy(ones_v,
                                     acc_sh.at[idx_v.at[w * wave + j]],
                                     sem, add=True)
            for j in range(wave):
                @pl.when(c0 + w * wave + j < n_eff)
                def _():
                    pltpu.make_async_copy(
                        ones_v, acc_sh.at[idx_v.at[0]], sem).wait()

        plsc.subcore_barrier()

        def copy_out(out_hbm):
            pltpu.sync_copy(acc_sh.at[pl.ds(row0, rows_main)],
                            zero_v.at[pl.ds(0, rows_main)])
            pltpu.sync_copy(zero_v.at[pl.ds(0, rows_main)],
                            out_hbm.at[pl.ds(row0, rows_main)])

            @pl.when(sid == _NS - 1)
            def _():
                pltpu.sync_copy(
                    acc_sh.at[pl.ds(_NS * rows_main, rows_tail)],
                    zero_v.at[pl.ds(rows_main, rows_tail)],
                )
                pltpu.sync_copy(
                    zero_v.at[pl.ds(rows_main, rows_tail)],
                    out_hbm.at[pl.ds(_NS * rows_main, rows_tail)],
                )

        @pl.when(cid == 0)
        def _():
            copy_out(out0_hbm)

        @pl.when(cid == 1)
        def _():
            copy_out(out1_hbm)

    return k(dst2)


def _sc_aggregate(g, src_p, dst_p, zeros2, n_nodes, e_real):
    d = g.shape[1]
    e_pad = src_p.shape[0]
    n_chunks = e_pad // _CH
    per_tile = n_chunks // (_NC * _NS)
    assert per_tile % _NBUF == 0
    rows_main, rows_tail = _row_split(n_nodes)
    n_acc = n_nodes + 8

    mesh = plsc.VectorSubcoreMesh(core_axis_name="c", subcore_axis_name="s")

    @functools.partial(
        pl.kernel,
        out_type=jax.ShapeDtypeStruct((_NC, n_nodes, d), jnp.float32),
        mesh=mesh,
        scratch_types=[
            pltpu.VMEM((2 * _NBUF, _CH), jnp.int32),
            pltpu.VMEM((2 * _NBUF, _CH), jnp.int32),
            pltpu.VMEM((_CH, d), jnp.float32),
            pltpu.VMEM((_CH, d), jnp.float32),
            pltpu.VMEM_SHARED((n_acc, d), jnp.float32),
            [pltpu.SemaphoreType.DMA] * _NBUF,
            [pltpu.SemaphoreType.DMA] * _NBUF,
            [pltpu.SemaphoreType.DMA] * (2 * _NBUF),
        ],
    )
    def k(g_hbm, src_hbm, dst_hbm, zeros_hbm, out_hbm, sidx_v, didx_v,
          rows0_v, rows1_v, acc_sh, sem_g, sem_s, sem_i):
        rows_bufs = (rows0_v, rows1_v)
        cid = lax.axis_index("c")
        sid = lax.axis_index("s")

        c0 = (cid * _NS + sid) * per_tile

        def load_idx(t, j, sem):
            base = (c0 + t) * _CH
            pltpu.async_copy(src_hbm.at[pl.ds(base, _CH)], sidx_v.at[j], sem)
            pltpu.async_copy(dst_hbm.at[pl.ds(base, _CH)], didx_v.at[j], sem)

        def wait_idx(j, sem):
            pltpu.make_async_copy(
                src_hbm.at[pl.ds(0, _CH)], sidx_v.at[j], sem).wait()
            pltpu.make_async_copy(
                dst_hbm.at[pl.ds(0, _CH)], didx_v.at[j], sem).wait()

        pltpu.sync_copy(zeros_hbm, rows0_v)
        row0 = sid * rows_main
        nz_full = rows_main // _CH
        z_rem = rows_main - nz_full * _CH

        @pl.loop(0, nz_full)
        def _(j):
            pltpu.sync_copy(rows0_v,
                            acc_sh.at[pl.ds(row0 + j * _CH, _CH)])

        pltpu.sync_copy(rows0_v.at[pl.ds(0, z_rem)],
                        acc_sh.at[pl.ds(row0 + nz_full * _CH, z_rem)])

        @pl.when(sid == _NS - 1)
        def _():
            pltpu.sync_copy(
                rows0_v.at[pl.ds(0, rows_tail + 8)],
                acc_sh.at[pl.ds(_NS * rows_main, rows_tail + 8)],
            )

        n_eff = -(-e_real // _CH)

        nslot = 2 * _NBUF
        for s in range(_NBUF):
            @pl.when(c0 + s < n_eff)
            def _():
                load_idx(s, s, sem_i[s])

        plsc.subcore_barrier()

        assert per_tile % nslot == 0

        @pl.loop(0, per_tile // nslot)
        def _(i):
            t = i * nslot
            for jj in range(nslot):
                tj = t + jj
                buf = jj % _NBUF
                prev_buf = (jj - 1) % _NBUF
                refill = (jj + _NBUF) % nslot

                @pl.when(c0 + tj < n_eff)
                def _():
                    wait_idx(jj, sem_i[jj])
                    pltpu.async_copy(g_hbm.at[sidx_v.at[jj]],
                                     rows_bufs[buf], sem_g[buf]).wait()

                    @pl.when(tj >= 1)
                    def _():
                        pltpu.make_async_copy(
                            rows_bufs[prev_buf],
                            acc_sh.at[didx_v.at[0]], sem_s[0]).wait()

                    pltpu.async_copy(rows_bufs[buf],
                                     acc_sh.at[didx_v.at[jj]],
                                     sem_s[0], add=True)

                    @pl.when(jnp.logical_and(tj + _NBUF < per_tile,
                                             c0 + tj + _NBUF < n_eff))
                    def _():
                        load_idx(tj + _NBUF, refill, sem_i[refill])

        @pl.when(c0 < n_eff)
        def _():
            pltpu.make_async_copy(
                rows_bufs[0],
                acc_sh.at[didx_v.at[0]], sem_s[0]).wait()

        plsc.subcore_barrier()
        pltpu.sync_copy(
            acc_sh.at[pl.ds(row0, rows_main)],
            out_hbm.at[cid].at[pl.ds(row0, rows_main)],
        )

        @pl.when(sid == _NS - 1)
        def _():
            pltpu.sync_copy(
                acc_sh.at[pl.ds(_NS * rows_main, rows_tail)],
                out_hbm.at[cid].at[pl.ds(_NS * rows_main, rows_tail)],
            )

    return k(g, src_p, dst_p, zeros2)


def _dinv_from_parts(d0_ref, d1_ref):
    deg = d0_ref[...] + d1_ref[...]
    return jnp.where(deg > 0, lax.rsqrt(jnp.maximum(deg, 1e-12)), 0.0)


def _tc_prep(x, w, d0c, d1c):
    n, d = x.shape
    br = 2000
    assert n % br == 0

    def body(x_ref, w_ref, d0_ref, d1_ref, g_ref):
        dinv = _dinv_from_parts(d0_ref, d1_ref)
        h = jnp.dot(x_ref[...], w_ref[...], preferred_element_type=jnp.float32)
        g_ref[...] = h * dinv

    return pl.pallas_call(
        body,
        grid=(n // br,),
        in_specs=[
            pl.BlockSpec((br, d), lambda i: (i, 0)),
            pl.BlockSpec((d, d), lambda i: (0, 0)),
            pl.BlockSpec((br, 1), lambda i: (i, 0)),
            pl.BlockSpec((br, 1), lambda i: (i, 0)),
        ],
        out_specs=pl.BlockSpec((br, d), lambda i: (i, 0)),
        out_shape=jax.ShapeDtypeStruct((n, d), jnp.float32),
    )(x, w, d0c, d1c)


def _tc_finish(accp, d0c, d1c, b2):
    n, d = accp.shape[1], accp.shape[2]
    br = 2000
    assert n % br == 0

    def body(a_ref, d0_ref, d1_ref, b_ref, o_ref):
        dinv = _dinv_from_parts(d0_ref, d1_ref)
        o_ref[...] = (a_ref[0] + a_ref[1]) * dinv + b_ref[...]

    return pl.pallas_call(
        body,
        grid=(n // br,),
        in_specs=[
            pl.BlockSpec((_NC, br, d), lambda i: (0, i, 0)),
            pl.BlockSpec((br, 1), lambda i: (i, 0)),
            pl.BlockSpec((br, 1), lambda i: (i, 0)),
            pl.BlockSpec((1, d), lambda i: (0, 0)),
        ],
        out_specs=pl.BlockSpec((br, d), lambda i: (i, 0)),
        out_shape=jax.ShapeDtypeStruct((n, d), jnp.float32),
    )(accp, d0c, d1c, b2)


def kernel(x, edge_index, W, b):
    n, d = x.shape
    e = edge_index.shape[1]
    ei = edge_index.astype(jnp.int32)
    n_chunks, _ = _pad_chunks(e)
    pad = n_chunks * _CH - e
    src_p = jnp.concatenate([ei[0], jnp.zeros((pad,), jnp.int32)])
    dst_p = jnp.concatenate([ei[1], jnp.full((pad,), n, jnp.int32)])
    dst2 = dst_p.reshape(n_chunks, _CH)
    zeros2 = jnp.zeros((_CH, d), jnp.float32)

    deg0, deg1 = _sc_degree(dst2, n, e)
    d0c = deg0.reshape(n, 1)
    d1c = deg1.reshape(n, 1)
    g = _tc_prep(x, W, d0c, d1c)
    accp = _sc_aggregate(g, src_p, dst_p, zeros2, n, e)
    return _tc_finish(accp, d0c, d1c, b.reshape(1, d))
